# baseline (speedup 1.0000x reference)
import jax
import jax.numpy as jnp
from jax.experimental import pallas as pl
from jax.experimental.pallas import tpu as pltpu

_NBG = 2
_N = 4096
_FT = 256
_HID = 64
_BM = 512
_NBLK = _N // _BM


def _dotT(u, v):
    return jax.lax.dot_general(u, v, (((1,), (1,)), ((), ())),
                               preferred_element_type=jnp.float32)


def _fused_kernel(f_ref, s_ref, wg_ref, a_ref, wd_ref, bd_ref, wa_ref, ba_ref,
                  h_in_ref, sb1_ref, sb2_ref,
                  log_ref, reg_ref, seq_scr, hh_scr, csum_scr, racc_scr):
    i = pl.program_id(0)
    j = pl.program_id(1)

    @pl.when((i == 0) & (j == 0))
    def _seq_graph0():
        w = wg_ref[0]
        seq_scr[0, :, :_HID] = jnp.dot(f_ref[0], w, preferred_element_type=jnp.float32)
        seq_scr[0, :, _HID:] = jnp.dot(s_ref[0], w, preferred_element_type=jnp.float32)

    @pl.when((i == 0) & (j >= 1))
    def _seq_graph1_chunk():
        w = wg_ref[0]
        rows = pl.ds((j - 1) * _BM, _BM)
        seq_scr[1, rows, :_HID] = jnp.dot(f_ref[0, rows, :], w,
                                          preferred_element_type=jnp.float32)
        seq_scr[1, rows, _HID:] = jnp.dot(s_ref[0, rows, :], w,
                                          preferred_element_type=jnp.float32)

    @pl.when((i == 0) & (j == _NBLK - 1))
    def _seq_graph1_last_chunk():
        w = wg_ref[0]
        rows = pl.ds((_NBLK - 1) * _BM, _BM)
        seq_scr[1, rows, :_HID] = jnp.dot(f_ref[0, rows, :], w,
                                          preferred_element_type=jnp.float32)
        seq_scr[1, rows, _HID:] = jnp.dot(s_ref[0, rows, :], w,
                                          preferred_element_type=jnp.float32)

    h = jax.nn.relu(jnp.dot(a_ref[0], seq_scr[pl.ds(i, 1), :, :][0],
                            preferred_element_type=jnp.float32))
    hh_scr[pl.ds(i * _N + j * _BM, _BM), :] = h

    psum = jnp.sum(h[:, :_HID], axis=0, keepdims=True)[None]
    prev = csum_scr[pl.ds(i, 1), :, :]
    csum_scr[pl.ds(i, 1), :, :] = jnp.where(j == 0, psum, prev + psum)

    @pl.when(i == _NBG - 1)
    def _reg_partial():
        rows = pl.ds(j * _BM, _BM)
        h1a = (hh_scr[rows, :_HID] + h[:, :_HID]) * 0.5
        h2a = (hh_scr[rows, _HID:] + h[:, _HID:]) * 0.5
        hb = h_in_ref[rows, :]
        pr = jnp.sum((h2a - h1a) * (2.0 * hb - h1a - h2a), keepdims=True)
        racc_scr[:, :] = jnp.where(j == 0, pr, racc_scr[:, :] + pr)

    @pl.when((i == _NBG - 1) & (j == 0))
    def _graph0_logits():
        c0 = jax.nn.sigmoid(csum_scr[0, :, :] * (1.0 / _N))
        v0 = _dotT(c0, wd_ref[:, :])
        sc1 = _dotT(v0, hh_scr[:_N, :_HID]) + bd_ref[0, 0] + sb1_ref[:, :]
        sc2 = _dotT(v0, hh_scr[:_N, _HID:]) + bd_ref[0, 0] + sb2_ref[:, :]
        log_ref[0] = jnp.concatenate([sc1, sc2], axis=0)

    @pl.when((i == _NBG - 1) & (j == _NBLK - 1))
    def _epilogue():
        wd = wd_ref[:, :]
        bd = bd_ref[0, 0]
        wa = wa_ref[:, :]
        ba = ba_ref[0, 0]
        sb1 = sb1_ref[:, :]
        sb2 = sb2_ref[:, :]
        h1g0 = hh_scr[:_N, :_HID]
        h2g0 = hh_scr[:_N, _HID:]
        h1g1 = hh_scr[_N:, :_HID]
        h2g1 = hh_scr[_N:, _HID:]
        c1 = jax.nn.sigmoid(csum_scr[1, :, :] * (1.0 / _N))
        v1 = _dotT(c1, wd)
        sc1 = _dotT(v1, h1g1) + bd + sb1
        sc2 = _dotT(v1, h2g1) + bd + sb2
        log_ref[1] = jnp.concatenate([sc1, sc2], axis=0)
        ca = jax.nn.sigmoid((csum_scr[0, :, :] + csum_scr[1, :, :])
                            * (0.5 / _N))
        va = _dotT(ca, wa)
        sca1 = (_dotT(va, h1g0) + _dotT(va, h1g1)) * 0.5 + ba + sb1
        sca2 = (_dotT(va, h2g0) + _dotT(va, h2g1)) * 0.5 + ba + sb2
        log_ref[2] = jnp.concatenate([sca1, sca2], axis=0)
        reg_ref[:, :] = racc_scr[:, :]


def kernel(feature, adj, shuf, sparse, msk, samp_bias1, samp_bias2,
           W_gcn, W_disc, b_disc, W_discAll, b_discAll, H):
    f = feature.reshape(_NBG, _N, _FT)
    a = adj.reshape(_NBG, _N, _N)
    s = shuf.reshape(_NBG, _N, _FT)
    h0 = H.reshape(_N, _HID)
    bd = b_disc.reshape(1, 1)
    ba = b_discAll.reshape(1, 1)

    log, reg = pl.pallas_call(
        _fused_kernel,
        grid=(_NBG, _NBLK),
        in_specs=[
            pl.BlockSpec((1, _N, _FT),
                         lambda i, j: (jnp.where((i == 0) & (j == 0), 0, 1), 0, 0)),
            pl.BlockSpec((1, _N, _FT),
                         lambda i, j: (jnp.where((i == 0) & (j == 0), 0, 1), 0, 0)),
            pl.BlockSpec((1, _FT, _HID),
                         lambda i, j: (jnp.where((i == 0) & (j == 0), 0, 1), 0, 0)),
            pl.BlockSpec((1, _BM, _N), lambda i, j: (i, j, 0)),
            pl.BlockSpec((_HID, _HID), lambda i, j: (0, 0)),
            pl.BlockSpec((1, 1), lambda i, j: (0, 0)),
            pl.BlockSpec((_HID, _HID), lambda i, j: (0, 0)),
            pl.BlockSpec((1, 1), lambda i, j: (0, 0)),
            pl.BlockSpec((_N, _HID), lambda i, j: (0, 0)),
            pl.BlockSpec((1, _N), lambda i, j: (0, 0)),
            pl.BlockSpec((1, _N), lambda i, j: (0, 0)),
        ],
        out_specs=[
            pl.BlockSpec((3, 2, _N), lambda i, j: (0, 0, 0)),
            pl.BlockSpec((1, 1), lambda i, j: (0, 0)),
        ],
        out_shape=[
            jax.ShapeDtypeStruct((3, 2, _N), jnp.float32),
            jax.ShapeDtypeStruct((1, 1), jnp.float32),
        ],
        scratch_shapes=[
            pltpu.VMEM((_NBG, _N, 2 * _HID), jnp.float32),
            pltpu.VMEM((_NBG * _N, 2 * _HID), jnp.float32),
            pltpu.VMEM((_NBG, 1, _HID), jnp.float32),
            pltpu.VMEM((1, 1), jnp.float32),
        ],
        compiler_params=pltpu.CompilerParams(
            dimension_semantics=("arbitrary", "arbitrary"),
            vmem_limit_bytes=100 * 1024 * 1024,
        ),
    )(f, s, W_gcn, a, W_disc, bd, W_discAll, ba, h0, samp_bias1, samp_bias2)

    logits0 = log[0].reshape(1, 2 * _N)
    logits1 = log[1].reshape(1, 2 * _N)
    logits2 = log[2].reshape(1, 2 * _N)
    reg_loss = reg.reshape(())
    return (logits0, logits1, logits2, reg_loss)
